# Initial kernel scaffold; baseline (speedup 1.0000x reference)
import jax
import jax.numpy as jnp
from jax.experimental import pallas as pl


def kernel(x, edge_index, W_gcn, b_gcn, W_lin, b_lin):
    raise NotImplementedError("write your pallas kernel here")


# speedup vs baseline: 35.4806x; 35.4806x over previous
import functools

import jax
import jax.numpy as jnp
from jax import lax
from jax.experimental import pallas as pl
from jax.experimental.pallas import tpu as pltpu
from jax.experimental.pallas import tpu_sc as plsc

N_NODES = 10000
N_EDGES = 320000
D_IN = 128
HID = 64
NODES_PER_GRAPH = 100
N_GRAPHS = N_NODES // NODES_PER_GRAPH
N_OUT = 10

NC = 2
NS = 16
NW = NC * NS

CHUNK = 128
N_ROWS = 2560
E_PAD = N_ROWS * CHUNK
ROWS_PER_WORKER = N_ROWS // NW
N_PAD = 12288
SLICE_PER_TILE = N_PAD // NS
DIS_PER_WORKER = N_PAD // NW

_mesh = plsc.VectorSubcoreMesh(core_axis_name="c", subcore_axis_name="s")
_sc_params = pltpu.CompilerParams(use_tc_tiling_on_sc=False)


def _rsqrt16(d):
    i = lax.bitcast_convert_type(d, jnp.int32)
    i = jnp.int32(0x5F3759DF) - (i >> 1)
    y = lax.bitcast_convert_type(i, jnp.float32)
    for _ in range(3):
        y = y * (1.5 - 0.5 * d * y * y)
    return y


@functools.partial(
    pl.kernel,
    out_type=jax.ShapeDtypeStruct((N_PAD,), jnp.float32),
    mesh=_mesh,
    scratch_types=[
        pltpu.VMEM((ROWS_PER_WORKER, CHUNK), jnp.int32),
        pltpu.VMEM((CHUNK,), jnp.float32),
        pltpu.VMEM((SLICE_PER_TILE,), jnp.float32),
        pltpu.VMEM_SHARED((N_PAD,), jnp.float32),
    ],
    compiler_params=_sc_params,
)
def _deg_dis(dst_hbm, dis_hbm, dst_v, ones_v, buf_v, deg_sh):
    c = lax.axis_index("c")
    s = lax.axis_index("s")
    for i in range(CHUNK // 16):
        ones_v[pl.ds(i * 16, 16)] = jnp.full((16,), 1.0, jnp.float32)
    for i in range(SLICE_PER_TILE // 16):
        buf_v[pl.ds(i * 16, 16)] = jnp.zeros((16,), jnp.float32)
    pltpu.sync_copy(buf_v, deg_sh.at[pl.ds(s * SLICE_PER_TILE, SLICE_PER_TILE)])
    plsc.subcore_barrier()
    for g in range(2):
        pltpu.sync_copy(dst_hbm.at[2 * s + g], dst_v)

        def body(j, carry):
            pltpu.sync_copy(ones_v, deg_sh.at[dst_v.at[j]], add=True)
            return carry

        lax.fori_loop(0, ROWS_PER_WORKER, body, 0)
    plsc.subcore_barrier()
    w = c * NS + s
    pltpu.sync_copy(deg_sh.at[pl.ds(w * DIS_PER_WORKER, DIS_PER_WORKER)],
                    buf_v.at[pl.ds(0, DIS_PER_WORKER)])
    for i in range(DIS_PER_WORKER // 16):
        d = buf_v[pl.ds(i * 16, 16)] + 1.0
        buf_v[pl.ds(i * 16, 16)] = _rsqrt16(d)
    pltpu.sync_copy(buf_v.at[pl.ds(0, DIS_PER_WORKER)],
                    dis_hbm.at[pl.ds(w * DIS_PER_WORKER, DIS_PER_WORKER)])


@functools.partial(
    pl.kernel,
    out_type=jax.ShapeDtypeStruct((NC * N_PAD, HID), jnp.float32),
    mesh=_mesh,
    scratch_types=[
        pltpu.VMEM((ROWS_PER_WORKER, CHUNK), jnp.int32),
        pltpu.VMEM((ROWS_PER_WORKER, CHUNK), jnp.int32),
        pltpu.VMEM((CHUNK, HID), jnp.float32),
        pltpu.VMEM_SHARED((N_PAD, HID), jnp.float32),
        pltpu.SemaphoreType.DMA,
    ],
    compiler_params=_sc_params,
)
def _agg(src_hbm, dst_hbm, hs_hbm, out_hbm, src_v, dst_v, rows_v, agg_sh, sem):
    c = lax.axis_index("c")
    s = lax.axis_index("s")
    w = c * NS + s

    def zbody(r, carry):
        for k in range(HID // 16):
            rows_v[r, pl.ds(k * 16, 16)] = jnp.zeros((16,), jnp.float32)
        return carry

    lax.fori_loop(0, CHUNK, zbody, 0)
    for i in range(SLICE_PER_TILE // CHUNK):
        pltpu.sync_copy(rows_v, agg_sh.at[pl.ds(s * SLICE_PER_TILE + i * CHUNK, CHUNK)])
    plsc.subcore_barrier()

    pltpu.sync_copy(src_hbm.at[w], src_v)
    pltpu.sync_copy(dst_hbm.at[w], dst_v)

    def body(j, carry):
        pltpu.async_copy(hs_hbm.at[src_v.at[j]], rows_v, sem).wait()
        pltpu.sync_copy(rows_v, agg_sh.at[dst_v.at[j]], add=True)
        return carry

    lax.fori_loop(0, ROWS_PER_WORKER, body, 0)
    plsc.subcore_barrier()
    pltpu.sync_copy(
        agg_sh.at[pl.ds(s * SLICE_PER_TILE, SLICE_PER_TILE)],
        out_hbm.at[pl.ds(c * N_PAD + s * SLICE_PER_TILE, SLICE_PER_TILE)])


def _prescale_body(x_ref, w_ref, dis_ref, hs_ref):
    h = jnp.dot(x_ref[...], w_ref[...], preferred_element_type=jnp.float32)
    hs_ref[...] = h * dis_ref[...]


_prescale = pl.pallas_call(
    _prescale_body,
    out_shape=jax.ShapeDtypeStruct((N_NODES, HID), jnp.float32),
)


def _combine_body(sp_ref, hs_ref, dis_ref, b_ref, act_ref):
    ssum = sp_ref[0:N_NODES, :] + sp_ref[N_PAD:N_PAD + N_NODES, :]
    a = (ssum + hs_ref[...]) * dis_ref[...] + b_ref[...]
    act_ref[...] = jnp.maximum(a, 0.0)


_combine = pl.pallas_call(
    _combine_body,
    out_shape=jax.ShapeDtypeStruct((N_NODES, HID), jnp.float32),
)


def _head_body(a_ref, w_ref, b_ref, o_ref):
    logits = jnp.dot(a_ref[...], w_ref[...],
                     preferred_element_type=jnp.float32) + b_ref[...]
    m = jnp.max(logits, axis=1, keepdims=True)
    lse = jnp.log(jnp.sum(jnp.exp(logits - m), axis=1, keepdims=True)) + m
    o_ref[...] = logits - lse


_head = pl.pallas_call(
    _head_body,
    out_shape=jax.ShapeDtypeStruct((N_GRAPHS, N_OUT), jnp.float32),
)


def kernel(x, edge_index, W_gcn, b_gcn, W_lin, b_lin):
    n_extra = E_PAD - N_EDGES
    pad_src = jnp.arange(n_extra, dtype=jnp.int32) % N_NODES
    pad_dst = N_NODES + jnp.arange(n_extra, dtype=jnp.int32) % (N_PAD - N_NODES)
    src3d = jnp.concatenate([edge_index[0], pad_src]).reshape(
        NW, ROWS_PER_WORKER, CHUNK)
    dst3d = jnp.concatenate([edge_index[1], pad_dst]).reshape(
        NW, ROWS_PER_WORKER, CHUNK)
    dis = _deg_dis(dst3d)
    dis_col = dis[:N_NODES].reshape(N_NODES, 1)
    hs = _prescale(x, W_gcn, dis_col)
    s_part = _agg(src3d, dst3d, hs)
    act = _combine(s_part, hs, dis_col, b_gcn)
    act2 = act.reshape(N_GRAPHS, HID * NODES_PER_GRAPH)
    return _head(act2, W_lin, b_lin)


# speedup vs baseline: 45.7206x; 1.2886x over previous
import functools

import jax
import jax.numpy as jnp
from jax import lax
from jax.experimental import pallas as pl
from jax.experimental.pallas import tpu as pltpu
from jax.experimental.pallas import tpu_sc as plsc

N_NODES = 10000
N_EDGES = 320000
D_IN = 128
HID = 64
NODES_PER_GRAPH = 100
N_GRAPHS = N_NODES // NODES_PER_GRAPH
N_OUT = 10

NC = 2
NS = 16
NW = NC * NS

CHUNK = 128
N_ROWS = 2560
E_PAD = N_ROWS * CHUNK
ROWS_PER_WORKER = N_ROWS // NW
N_PAD = 12288
SLICE_PER_TILE = N_PAD // NS
DIS_PER_WORKER = N_PAD // NW

_mesh = plsc.VectorSubcoreMesh(core_axis_name="c", subcore_axis_name="s")
_sc_params = pltpu.CompilerParams(use_tc_tiling_on_sc=False)


def _rsqrt16(d):
    i = lax.bitcast_convert_type(d, jnp.int32)
    i = jnp.int32(0x5F3759DF) - (i >> 1)
    y = lax.bitcast_convert_type(i, jnp.float32)
    for _ in range(3):
        y = y * (1.5 - 0.5 * d * y * y)
    return y


@functools.partial(
    pl.kernel,
    out_type=jax.ShapeDtypeStruct((N_PAD,), jnp.float32),
    mesh=_mesh,
    scratch_types=[
        pltpu.VMEM((ROWS_PER_WORKER, CHUNK), jnp.int32),
        pltpu.VMEM((CHUNK,), jnp.float32),
        pltpu.VMEM((SLICE_PER_TILE,), jnp.float32),
        pltpu.VMEM_SHARED((N_PAD,), jnp.float32),
    ],
    compiler_params=_sc_params,
)
def _deg_dis(dst_hbm, dis_hbm, dst_v, ones_v, buf_v, deg_sh):
    c = lax.axis_index("c")
    s = lax.axis_index("s")
    for i in range(CHUNK // 16):
        ones_v[pl.ds(i * 16, 16)] = jnp.full((16,), 1.0, jnp.float32)
    for i in range(SLICE_PER_TILE // 16):
        buf_v[pl.ds(i * 16, 16)] = jnp.zeros((16,), jnp.float32)
    pltpu.sync_copy(buf_v, deg_sh.at[pl.ds(s * SLICE_PER_TILE, SLICE_PER_TILE)])
    plsc.subcore_barrier()
    for g in range(2):
        pltpu.sync_copy(dst_hbm.at[2 * s + g], dst_v)

        def body(j, carry):
            pltpu.sync_copy(ones_v, deg_sh.at[dst_v.at[j]], add=True)
            return carry

        lax.fori_loop(0, ROWS_PER_WORKER, body, 0)
    plsc.subcore_barrier()
    w = c * NS + s
    pltpu.sync_copy(deg_sh.at[pl.ds(w * DIS_PER_WORKER, DIS_PER_WORKER)],
                    buf_v.at[pl.ds(0, DIS_PER_WORKER)])
    for i in range(DIS_PER_WORKER // 16):
        d = buf_v[pl.ds(i * 16, 16)] + 1.0
        buf_v[pl.ds(i * 16, 16)] = _rsqrt16(d)
    pltpu.sync_copy(buf_v.at[pl.ds(0, DIS_PER_WORKER)],
                    dis_hbm.at[pl.ds(w * DIS_PER_WORKER, DIS_PER_WORKER)])


@functools.partial(
    pl.kernel,
    out_type=jax.ShapeDtypeStruct((NC * N_PAD, HID), jnp.float32),
    mesh=_mesh,
    scratch_types=[
        pltpu.VMEM((ROWS_PER_WORKER, CHUNK), jnp.int32),
        pltpu.VMEM((ROWS_PER_WORKER, CHUNK), jnp.int32),
        pltpu.VMEM((2, CHUNK, HID), jnp.float32),
        pltpu.VMEM_SHARED((N_PAD, HID), jnp.float32),
        pltpu.SemaphoreType.DMA,
        pltpu.SemaphoreType.DMA,
    ],
    compiler_params=_sc_params,
)
def _agg(src_hbm, dst_hbm, hs_hbm, out_hbm, src_v, dst_v, rows_v, agg_sh,
         sem0, sem1):
    c = lax.axis_index("c")
    s = lax.axis_index("s")
    w = c * NS + s

    def zbody(r, carry):
        for k in range(HID // 16):
            rows_v[0, r, pl.ds(k * 16, 16)] = jnp.zeros((16,), jnp.float32)
        return carry

    lax.fori_loop(0, CHUNK, zbody, 0)
    for i in range(SLICE_PER_TILE // CHUNK):
        pltpu.sync_copy(rows_v.at[0],
                        agg_sh.at[pl.ds(s * SLICE_PER_TILE + i * CHUNK, CHUNK)])
    plsc.subcore_barrier()

    pltpu.sync_copy(src_hbm.at[w], src_v)
    pltpu.sync_copy(dst_hbm.at[w], dst_v)

    def gather(j, b, sem):
        return pltpu.async_copy(hs_hbm.at[src_v.at[j]], rows_v.at[b], sem)

    gather(0, 0, sem0)

    def body(t, carry):
        j0 = 2 * t
        gather(j0 + 1, 1, sem1)
        pltpu.make_async_copy(hs_hbm.at[src_v.at[j0]], rows_v.at[0], sem0).wait()
        pltpu.sync_copy(rows_v.at[0], agg_sh.at[dst_v.at[j0]], add=True)

        @pl.when(t < ROWS_PER_WORKER // 2 - 1)
        def _():
            gather(j0 + 2, 0, sem0)

        pltpu.make_async_copy(hs_hbm.at[src_v.at[j0 + 1]], rows_v.at[1],
                              sem1).wait()
        pltpu.sync_copy(rows_v.at[1], agg_sh.at[dst_v.at[j0 + 1]], add=True)
        return carry

    lax.fori_loop(0, ROWS_PER_WORKER // 2, body, 0)
    plsc.subcore_barrier()
    pltpu.sync_copy(
        agg_sh.at[pl.ds(s * SLICE_PER_TILE, SLICE_PER_TILE)],
        out_hbm.at[pl.ds(c * N_PAD + s * SLICE_PER_TILE, SLICE_PER_TILE)])


def _prescale_body(x_ref, w_ref, dis_ref, hs_ref):
    h = jnp.dot(x_ref[...], w_ref[...], preferred_element_type=jnp.float32)
    hs_ref[...] = h * dis_ref[...]


_prescale = pl.pallas_call(
    _prescale_body,
    out_shape=jax.ShapeDtypeStruct((N_NODES, HID), jnp.float32),
)


def _combine_body(sp_ref, hs_ref, dis_ref, b_ref, act_ref):
    ssum = sp_ref[0:N_NODES, :] + sp_ref[N_PAD:N_PAD + N_NODES, :]
    a = (ssum + hs_ref[...]) * dis_ref[...] + b_ref[...]
    act_ref[...] = jnp.maximum(a, 0.0)


_combine = pl.pallas_call(
    _combine_body,
    out_shape=jax.ShapeDtypeStruct((N_NODES, HID), jnp.float32),
)


def _head_body(a_ref, w_ref, b_ref, o_ref):
    logits = jnp.dot(a_ref[...], w_ref[...],
                     preferred_element_type=jnp.float32) + b_ref[...]
    m = jnp.max(logits, axis=1, keepdims=True)
    lse = jnp.log(jnp.sum(jnp.exp(logits - m), axis=1, keepdims=True)) + m
    o_ref[...] = logits - lse


_head = pl.pallas_call(
    _head_body,
    out_shape=jax.ShapeDtypeStruct((N_GRAPHS, N_OUT), jnp.float32),
)


def kernel(x, edge_index, W_gcn, b_gcn, W_lin, b_lin):
    n_extra = E_PAD - N_EDGES
    pad_src = jnp.arange(n_extra, dtype=jnp.int32) % N_NODES
    pad_dst = N_NODES + jnp.arange(n_extra, dtype=jnp.int32) % (N_PAD - N_NODES)
    src3d = jnp.concatenate([edge_index[0], pad_src]).reshape(
        NW, ROWS_PER_WORKER, CHUNK)
    dst3d = jnp.concatenate([edge_index[1], pad_dst]).reshape(
        NW, ROWS_PER_WORKER, CHUNK)
    dis = _deg_dis(dst3d)
    dis_col = dis[:N_NODES].reshape(N_NODES, 1)
    hs = _prescale(x, W_gcn, dis_col)
    s_part = _agg(src3d, dst3d, hs)
    act = _combine(s_part, hs, dis_col, b_gcn)
    act2 = act.reshape(N_GRAPHS, HID * NODES_PER_GRAPH)
    return _head(act2, W_lin, b_lin)


# speedup vs baseline: 50.3301x; 1.1008x over previous
import functools

import jax
import jax.numpy as jnp
from jax import lax
from jax.experimental import pallas as pl
from jax.experimental.pallas import tpu as pltpu
from jax.experimental.pallas import tpu_sc as plsc

N_NODES = 10000
N_EDGES = 320000
D_IN = 128
HID = 64
NODES_PER_GRAPH = 100
N_GRAPHS = N_NODES // NODES_PER_GRAPH
N_OUT = 10

NC = 2
NS = 16
NW = NC * NS

CHUNK = 128
N_ROWS = 2560
E_PAD = N_ROWS * CHUNK
ROWS_PER_WORKER = N_ROWS // NW
N_PAD = 12288
SLICE_PER_TILE = N_PAD // NS
DIS_PER_WORKER = N_PAD // NW

_mesh = plsc.VectorSubcoreMesh(core_axis_name="c", subcore_axis_name="s")
_sc_params = pltpu.CompilerParams(use_tc_tiling_on_sc=False)


def _rsqrt16(d):
    i = lax.bitcast_convert_type(d, jnp.int32)
    i = jnp.int32(0x5F3759DF) - (i >> 1)
    y = lax.bitcast_convert_type(i, jnp.float32)
    for _ in range(3):
        y = y * (1.5 - 0.5 * d * y * y)
    return y


@functools.partial(
    pl.kernel,
    out_type=jax.ShapeDtypeStruct((N_PAD,), jnp.float32),
    mesh=_mesh,
    scratch_types=[
        pltpu.VMEM((ROWS_PER_WORKER, CHUNK), jnp.int32),
        pltpu.VMEM((CHUNK,), jnp.float32),
        pltpu.VMEM((SLICE_PER_TILE,), jnp.float32),
        pltpu.VMEM_SHARED((N_PAD,), jnp.float32),
    ],
    compiler_params=_sc_params,
)
def _deg_dis(dst_hbm, dis_hbm, dst_v, ones_v, buf_v, deg_sh):
    c = lax.axis_index("c")
    s = lax.axis_index("s")
    for i in range(CHUNK // 16):
        ones_v[pl.ds(i * 16, 16)] = jnp.full((16,), 1.0, jnp.float32)
    for i in range(SLICE_PER_TILE // 16):
        buf_v[pl.ds(i * 16, 16)] = jnp.zeros((16,), jnp.float32)
    pltpu.sync_copy(buf_v, deg_sh.at[pl.ds(s * SLICE_PER_TILE, SLICE_PER_TILE)])
    plsc.subcore_barrier()
    for g in range(2):
        pltpu.sync_copy(dst_hbm.at[2 * s + g], dst_v)

        def body(j, carry):
            pltpu.sync_copy(ones_v, deg_sh.at[dst_v.at[j]], add=True)
            return carry

        lax.fori_loop(0, ROWS_PER_WORKER, body, 0)
    plsc.subcore_barrier()
    w = c * NS + s
    pltpu.sync_copy(deg_sh.at[pl.ds(w * DIS_PER_WORKER, DIS_PER_WORKER)],
                    buf_v.at[pl.ds(0, DIS_PER_WORKER)])
    for i in range(DIS_PER_WORKER // 16):
        d = buf_v[pl.ds(i * 16, 16)] + 1.0
        buf_v[pl.ds(i * 16, 16)] = _rsqrt16(d)
    pltpu.sync_copy(buf_v.at[pl.ds(0, DIS_PER_WORKER)],
                    dis_hbm.at[pl.ds(w * DIS_PER_WORKER, DIS_PER_WORKER)])


@functools.partial(
    pl.kernel,
    out_type=jax.ShapeDtypeStruct((NC * N_PAD, HID), jnp.float32),
    mesh=_mesh,
    scratch_types=[
        pltpu.VMEM((ROWS_PER_WORKER, CHUNK), jnp.int32),
        pltpu.VMEM((ROWS_PER_WORKER, CHUNK), jnp.int32),
        pltpu.VMEM((4, CHUNK, HID), jnp.float32),
        pltpu.VMEM_SHARED((N_PAD, HID), jnp.float32),
        pltpu.SemaphoreType.DMA,
        pltpu.SemaphoreType.DMA,
        pltpu.SemaphoreType.DMA,
        pltpu.SemaphoreType.DMA,
    ],
    compiler_params=_sc_params,
)
def _agg(src_hbm, dst_hbm, hs_hbm, out_hbm, src_v, dst_v, rows_v, agg_sh,
         sem0, sem1, sem2, sem3):
    c = lax.axis_index("c")
    s = lax.axis_index("s")
    w = c * NS + s

    def zbody(r, carry):
        for k in range(HID // 16):
            rows_v[0, r, pl.ds(k * 16, 16)] = jnp.zeros((16,), jnp.float32)
        return carry

    lax.fori_loop(0, CHUNK, zbody, 0)
    for i in range(SLICE_PER_TILE // CHUNK):
        pltpu.sync_copy(rows_v.at[0],
                        agg_sh.at[pl.ds(s * SLICE_PER_TILE + i * CHUNK, CHUNK)])
    plsc.subcore_barrier()

    pltpu.sync_copy(src_hbm.at[w], src_v)
    pltpu.sync_copy(dst_hbm.at[w], dst_v)

    sems = (sem0, sem1, sem2, sem3)

    def gather(j, b):
        pltpu.async_copy(hs_hbm.at[src_v.at[j]], rows_v.at[b], sems[b])

    for b in range(4):
        gather(b, b)

    n_t = ROWS_PER_WORKER // 4

    def body(t, carry):
        for b in range(4):
            j = 4 * t + b
            pltpu.make_async_copy(hs_hbm.at[src_v.at[j]], rows_v.at[b],
                                  sems[b]).wait()
            pltpu.sync_copy(rows_v.at[b], agg_sh.at[dst_v.at[j]], add=True)

            @pl.when(t < n_t - 1)
            def _():
                gather(j + 4, b)

        return carry

    lax.fori_loop(0, n_t, body, 0)
    plsc.subcore_barrier()
    pltpu.sync_copy(
        agg_sh.at[pl.ds(s * SLICE_PER_TILE, SLICE_PER_TILE)],
        out_hbm.at[pl.ds(c * N_PAD + s * SLICE_PER_TILE, SLICE_PER_TILE)])


def _prescale_body(x_ref, w_ref, dis_ref, hs_ref):
    h = jnp.dot(x_ref[...], w_ref[...], preferred_element_type=jnp.float32)
    hs_ref[...] = h * dis_ref[...]


_prescale = pl.pallas_call(
    _prescale_body,
    out_shape=jax.ShapeDtypeStruct((N_NODES, HID), jnp.float32),
)


def _combine_body(sp_ref, hs_ref, dis_ref, b_ref, act_ref):
    ssum = sp_ref[0:N_NODES, :] + sp_ref[N_PAD:N_PAD + N_NODES, :]
    a = (ssum + hs_ref[...]) * dis_ref[...] + b_ref[...]
    act_ref[...] = jnp.maximum(a, 0.0)


_combine = pl.pallas_call(
    _combine_body,
    out_shape=jax.ShapeDtypeStruct((N_NODES, HID), jnp.float32),
)


def _head_body(a_ref, w_ref, b_ref, o_ref):
    logits = jnp.dot(a_ref[...], w_ref[...],
                     preferred_element_type=jnp.float32) + b_ref[...]
    m = jnp.max(logits, axis=1, keepdims=True)
    lse = jnp.log(jnp.sum(jnp.exp(logits - m), axis=1, keepdims=True)) + m
    o_ref[...] = logits - lse


_head = pl.pallas_call(
    _head_body,
    out_shape=jax.ShapeDtypeStruct((N_GRAPHS, N_OUT), jnp.float32),
)


def kernel(x, edge_index, W_gcn, b_gcn, W_lin, b_lin):
    n_extra = E_PAD - N_EDGES
    pad_src = jnp.arange(n_extra, dtype=jnp.int32) % N_NODES
    pad_dst = N_NODES + jnp.arange(n_extra, dtype=jnp.int32) % (N_PAD - N_NODES)
    src3d = jnp.concatenate([edge_index[0], pad_src]).reshape(
        NW, ROWS_PER_WORKER, CHUNK)
    dst3d = jnp.concatenate([edge_index[1], pad_dst]).reshape(
        NW, ROWS_PER_WORKER, CHUNK)
    dis = _deg_dis(dst3d)
    dis_col = dis[:N_NODES].reshape(N_NODES, 1)
    hs = _prescale(x, W_gcn, dis_col)
    s_part = _agg(src3d, dst3d, hs)
    act = _combine(s_part, hs, dis_col, b_gcn)
    act2 = act.reshape(N_GRAPHS, HID * NODES_PER_GRAPH)
    return _head(act2, W_lin, b_lin)
